# baseline (speedup 1.0000x reference)
import functools

import jax
import jax.numpy as jnp
from jax import lax
from jax.experimental import pallas as pl
from jax.experimental.pallas import tpu as pltpu
from jax.experimental.pallas import tpu_sc as plsc

D = 32
NC = 2
NS = 16
NW = NC * NS
CB = 32


def _make_gather(nb, nh):
    blk = nb // NW
    nchunk = blk // CB
    mesh = plsc.VectorSubcoreMesh(core_axis_name="c", subcore_axis_name="s")

    @functools.partial(
        pl.kernel,
        mesh=mesh,
        out_type=jax.ShapeDtypeStruct((nb, nh, D), jnp.float32),
        scratch_types=[
            pltpu.VMEM((CB * nh,), jnp.int32),
            pltpu.VMEM((CB * nh,), jnp.int32),
            pltpu.VMEM((CB * nh, D), jnp.float32),
            pltpu.VMEM((CB * nh, D), jnp.float32),
            pltpu.SemaphoreType.DMA,
            pltpu.SemaphoreType.DMA,
            pltpu.SemaphoreType.DMA,
        ],
        compiler_params=pltpu.CompilerParams(use_tc_tiling_on_sc=False),
    )
    def gather_kernel(idx_hbm, table_hbm, out_hbm,
                      idx_v0, idx_v1, rows_v0, rows_v1, gsem, ssem0, ssem1):
        wid = lax.axis_index("s") * NC + lax.axis_index("c")
        b0 = wid * blk
        idx_vs = (idx_v0, idx_v1)
        rows_vs = (rows_v0, rows_v1)
        ssems = (ssem0, ssem1)
        gathers = [None, None]
        stores = [None, None]
        ch = CB * nh
        pltpu.sync_copy(idx_hbm.at[pl.ds(b0 * nh, ch)], idx_v0)
        gathers[0] = pltpu.async_copy(table_hbm.at[idx_v0], rows_v0, gsem)
        if nchunk > 1:
            pltpu.sync_copy(idx_hbm.at[pl.ds(b0 * nh + ch, ch)], idx_v1)
        for c in range(nchunk):
            p = c % 2
            gathers[p].wait()
            sts = []
            for b in range(CB):
                sts.append(pltpu.async_copy(
                    rows_vs[p].at[pl.ds(b * nh, nh)],
                    out_hbm.at[b0 + c * CB + b], ssems[p]))
            stores[p] = sts
            if c + 1 < nchunk:
                np_ = 1 - p
                if stores[np_] is not None:
                    for st in stores[np_]:
                        st.wait()
                gathers[np_] = pltpu.async_copy(
                    table_hbm.at[idx_vs[np_]], rows_vs[np_], gsem)
                if c + 2 < nchunk:
                    pltpu.sync_copy(
                        idx_hbm.at[pl.ds(b0 * nh + (c + 2) * ch, ch)], idx_vs[p])
        if nchunk > 1:
            for st in stores[(nchunk - 2) % 2]:
                st.wait()
        for st in stores[(nchunk - 1) % 2]:
            st.wait()

    return gather_kernel


def kernel(indices, weight):
    nb, nh = indices.shape
    flat = indices.reshape(-1).astype(jnp.int32)
    return _make_gather(nb, nh)(flat, weight)


# speedup vs baseline: 1.0826x; 1.0826x over previous
import functools

import jax
import jax.numpy as jnp
from jax import lax
from jax.experimental import pallas as pl
from jax.experimental.pallas import tpu as pltpu
from jax.experimental.pallas import tpu_sc as plsc

D = 32
NC = 2
NS = 16
NW = NC * NS
CH = 1280


def _make_gather(total):
    bpw = total // NW
    nchunk = bpw // CH
    mesh = plsc.VectorSubcoreMesh(core_axis_name="c", subcore_axis_name="s")

    @functools.partial(
        pl.kernel,
        mesh=mesh,
        out_type=jax.ShapeDtypeStruct((total, D), jnp.float32),
        scratch_types=[
            pltpu.VMEM((CH,), jnp.int32),
            pltpu.VMEM((CH,), jnp.int32),
            pltpu.VMEM((CH, D), jnp.float32),
            pltpu.VMEM((CH, D), jnp.float32),
            pltpu.SemaphoreType.DMA,
            pltpu.SemaphoreType.DMA,
            pltpu.SemaphoreType.DMA,
        ],
        compiler_params=pltpu.CompilerParams(use_tc_tiling_on_sc=False),
    )
    def gather_kernel(idx_hbm, table_hbm, out_hbm,
                      idx_v0, idx_v1, rows_v0, rows_v1, gsem, ssem0, ssem1):
        wid = lax.axis_index("s") * NC + lax.axis_index("c")
        base = wid * bpw
        idx_vs = (idx_v0, idx_v1)
        rows_vs = (rows_v0, rows_v1)
        ssems = (ssem0, ssem1)
        gathers = [None, None]
        stores = [None, None]
        pltpu.sync_copy(idx_hbm.at[pl.ds(base, CH)], idx_v0)
        gathers[0] = pltpu.async_copy(table_hbm.at[idx_v0], rows_v0, gsem)
        if nchunk > 1:
            pltpu.sync_copy(idx_hbm.at[pl.ds(base + CH, CH)], idx_v1)
        for c in range(nchunk):
            b = c % 2
            gathers[b].wait()
            stores[b] = pltpu.async_copy(
                rows_vs[b], out_hbm.at[pl.ds(base + c * CH, CH)], ssems[b])
            if c + 1 < nchunk:
                nb_ = 1 - b
                if stores[nb_] is not None:
                    stores[nb_].wait()
                gathers[nb_] = pltpu.async_copy(
                    table_hbm.at[idx_vs[nb_]], rows_vs[nb_], gsem)
                if c + 2 < nchunk:
                    pltpu.sync_copy(
                        idx_hbm.at[pl.ds(base + (c + 2) * CH, CH)], idx_vs[b])
        if nchunk > 1:
            stores[(nchunk - 2) % 2].wait()
        stores[(nchunk - 1) % 2].wait()

    return gather_kernel


def kernel(indices, weight):
    nb, nh = indices.shape
    flat = indices.T.reshape(-1).astype(jnp.int32)
    out = _make_gather(flat.shape[0])(flat, weight)
    return out.reshape(nh, nb, weight.shape[1]).transpose(1, 0, 2)
